# baseline (speedup 1.0000x reference)
import functools

import jax
import jax.numpy as jnp
from jax import lax
from jax.experimental import pallas as pl
from jax.experimental.pallas import tpu as pltpu
from jax.experimental.pallas import tpu_sc as plsc

N_NODES = 100000
NUM_TYPES = 11
MAP_PAD = 40

NW = 32
ROWS_W = 3136
ROWS_LAST = N_NODES - (NW - 1) * ROWS_W
GROUPS_FULL = ROWS_W // 16
GROUPS_LAST = ROWS_LAST // 16
WORDS_FULL = ROWS_W * NUM_TYPES
WORDS_LAST = ROWS_LAST * NUM_TYPES

_mesh = plsc.VectorSubcoreMesh(core_axis_name="c", subcore_axis_name="s")


@functools.partial(
    pl.kernel,
    mesh=_mesh,
    compiler_params=pltpu.CompilerParams(needs_layout_passes=False),
    out_type=jax.ShapeDtypeStruct((N_NODES * NUM_TYPES,), jnp.float32),
    scratch_types=[
        pltpu.VMEM((ROWS_W,), jnp.int32),
        pltpu.VMEM((MAP_PAD,), jnp.int32),
        pltpu.VMEM((WORDS_FULL,), jnp.float32),
    ],
)
def _onehot_sc(nt_hbm, map_hbm, out_hbm, nt_v, map_v, out_v):
    c = lax.axis_index("c")
    s = lax.axis_index("s")
    wid = s * 2 + c
    base = wid * ROWS_W
    last = wid == NW - 1

    pltpu.sync_copy(map_hbm, map_v)

    @pl.when(jnp.logical_not(last))
    def _():
        pltpu.sync_copy(nt_hbm.at[pl.ds(base, ROWS_W)], nt_v)

    @pl.when(last)
    def _():
        pltpu.sync_copy(
            nt_hbm.at[pl.ds(base, ROWS_LAST)], nt_v.at[pl.ds(0, ROWS_LAST)]
        )

    groups = jnp.where(last, GROUPS_LAST, GROUPS_FULL)
    iota16 = lax.iota(jnp.int32, 16)
    vals = jnp.full((16,), 0.25, jnp.float32)
    zeros = jnp.zeros((16,), jnp.float32)

    def body(g, carry):
        w0 = g * (16 * NUM_TYPES)
        for t in range(NUM_TYPES):
            out_v[pl.ds(w0 + t * 16, 16)] = zeros
        nt16 = nt_v[pl.ds(g * 16, 16)]
        idx16 = plsc.load_gather(map_v, [nt16])
        flat16 = (g * 16 + iota16) * NUM_TYPES + idx16
        plsc.store_scatter(out_v, [flat16], vals, mask=idx16 >= 0)
        return carry

    lax.fori_loop(0, groups, body, 0)

    @pl.when(jnp.logical_not(last))
    def _():
        pltpu.sync_copy(out_v, out_hbm.at[pl.ds(base * NUM_TYPES, WORDS_FULL)])

    @pl.when(last)
    def _():
        pltpu.sync_copy(
            out_v.at[pl.ds(0, WORDS_LAST)],
            out_hbm.at[pl.ds(base * NUM_TYPES, WORDS_LAST)],
        )


def kernel(node_type, pos, mapper):
    del pos
    nt = node_type.reshape(-1).astype(jnp.int32)
    mp = jnp.pad(mapper.astype(jnp.int32), (0, MAP_PAD - mapper.shape[0]),
                 constant_values=-1)
    out = _onehot_sc(nt, mp)
    return out.reshape(N_NODES, NUM_TYPES)
